# baseline (speedup 1.0000x reference)
import dataclasses
import functools

import jax
import jax.numpy as jnp
from jax import lax
from jax.experimental import pallas as pl
from jax.experimental.pallas import tpu as pltpu
from jax.experimental.pallas import tpu_sc as plsc

NC = 2
NS = 16
LANES = 16
NW = NC * NS

CB = 64


def _sc_compiler_params():
    cp = pltpu.CompilerParams()
    if "needs_layout_passes" in pltpu.CompilerParams.__dataclass_fields__:
        cp = dataclasses.replace(cp, needs_layout_passes=False)
    return cp


def _round_up(a: int, m: int) -> int:
    return ((a + m - 1) // m) * m


def _make_deg_kernel(n_pad: int, e_pad: int):
    e_per_w = e_pad // NW

    def body(dst_hbm, ew_hbm, out_hbm, dst_v, ew_v, acc_v):
        c = lax.axis_index("c")
        s = lax.axis_index("s")
        wid = c * NS + s
        base = wid * e_per_w

        zero16 = jnp.zeros((LANES,), jnp.float32)

        @pl.loop(0, n_pad, step=LANES)
        def _(i):
            acc_v[pl.ds(i, LANES)] = zero16

        pltpu.sync_copy(dst_hbm.at[pl.ds(base, e_per_w)], dst_v)
        pltpu.sync_copy(ew_hbm.at[pl.ds(base, e_per_w)], ew_v)

        lane = lax.iota(jnp.int32, LANES)

        @pl.loop(0, e_per_w, step=LANES)
        def _(i):
            idx = dst_v[pl.ds(i, LANES)]
            w = ew_v[pl.ds(i, LANES)]
            for l in range(LANES):
                plsc.addupdate_scatter(acc_v, [idx], w, mask=lane == l)

        pltpu.sync_copy(acc_v, out_hbm.at[pl.ds(wid * n_pad, n_pad)])

    return pl.kernel(
        body,
        out_type=jax.ShapeDtypeStruct((NW * n_pad,), jnp.float32),
        mesh=plsc.VectorSubcoreMesh(core_axis_name="c", subcore_axis_name="s"),
        scratch_types=[
            pltpu.VMEM((e_per_w,), jnp.int32),
            pltpu.VMEM((e_per_w,), jnp.float32),
            pltpu.VMEM((n_pad,), jnp.float32),
        ],
        compiler_params=_sc_compiler_params(),
    )


NBUF = 4
EBUF = 8


def _make_agg_kernel(n_pad: int, e_pad: int, d: int, q0: int, q1: int):
    rows_per_t = n_pad // NS
    n_wb = rows_per_t // CB
    assert q0 % EBUF == 0 and q1 % EBUF == 0 and min(q0, q1) >= EBUF
    assert (q0 + q1) * NS * CB == e_pad

    def body(g_hbm, ep_hbm, z_hbm, out_hbm, ebuf, rows_v, acc_sh, *sems):
        gsem = sems[:NBUF]
        ssem = sems[NBUF:2 * NBUF]
        esem = sems[2 * NBUF:]
        c = lax.axis_index("c")
        s = lax.axis_index("s")
        row0 = s * rows_per_t
        n_chunks = jnp.where(c == 0, q0, q1)
        chunk0 = jnp.where(c == 0, s * q0, NS * q0 + s * q1)

        pltpu.sync_copy(z_hbm.at[pl.ds(row0, rows_per_t)],
                        acc_sh.at[pl.ds(row0, rows_per_t)])

        plsc.subcore_barrier()

        def eload(k, eb):
            pltpu.async_copy(ep_hbm.at[chunk0 + k], ebuf.at[eb], esem[eb])

        def ewait(k, eb):
            pltpu.make_async_copy(ep_hbm.at[chunk0 + k], ebuf.at[eb],
                                  esem[eb]).wait()

        def gload(eb, rb):
            pltpu.async_copy(g_hbm.at[ebuf.at[eb, 0]], rows_v.at[rb],
                             gsem[rb])

        def gwait(eb, rb):
            pltpu.make_async_copy(g_hbm.at[ebuf.at[eb, 0]], rows_v.at[rb],
                                  gsem[rb]).wait()

        def swait(eb, rb):
            pltpu.make_async_copy(rows_v.at[rb], acc_sh.at[ebuf.at[eb, 1]],
                                  ssem[rb]).wait()

        for k in range(EBUF - 2):
            eload(k, k)
        for k in range(2):
            ewait(k, k)
            gload(k, k)

        @pl.loop(0, n_chunks, step=EBUF)
        def _(j0):
            for b in range(EBUF):
                j = j0 + b
                rb = b % NBUF
                buf = rows_v.at[rb]
                gwait(b, rb)

                @pl.loop(0, CB, step=LANES)
                def _(i):
                    wi = ebuf[b, 2, pl.ds(i, LANES)]
                    w16 = plsc.bitcast(wi, jnp.float32)
                    for l in range(LANES):
                        w = w16[l]
                        for jj in range(d // LANES):
                            sl = pl.ds(jj * LANES, LANES)
                            buf[i + l, sl] = buf[i + l, sl] * w

                pltpu.async_copy(buf, acc_sh.at[ebuf.at[b, 1]], ssem[rb],
                                 add=True)

                rb2 = (b + 2) % NBUF
                eb2 = (b + 2) % EBUF
                eb6 = (b + 6) % EBUF

                @pl.when(j >= 2)
                def _():
                    swait(eb2, rb2)

                @pl.when(j + 6 < n_chunks)
                def _():
                    eload(j + 6, eb6)

                @pl.when(j + 2 < n_chunks)
                def _():
                    ewait(j + 2, eb2)
                    gload(eb2, rb2)

        swait(EBUF - 2, NBUF - 2)
        swait(EBUF - 1, NBUF - 1)

        plsc.subcore_barrier()

        out_base = c * n_pad + row0
        pltpu.sync_copy(acc_sh.at[pl.ds(row0, rows_per_t)],
                        out_hbm.at[pl.ds(out_base, rows_per_t)])

    return pl.kernel(
        body,
        out_type=jax.ShapeDtypeStruct((NC * n_pad, d), jnp.float32),
        mesh=plsc.VectorSubcoreMesh(core_axis_name="c", subcore_axis_name="s"),
        scratch_types=[
            pltpu.VMEM((EBUF, 8, CB), jnp.int32),
            pltpu.VMEM((NBUF, CB, d), jnp.float32),
            pltpu.VMEM_SHARED((n_pad, d), jnp.float32),
        ] + [pltpu.SemaphoreType.DMA] * (2 * NBUF + EBUF),
        compiler_params=_sc_compiler_params(),
    )


def _mm_body(x_ref, w_ref, o_ref):
    o_ref[...] = lax.dot_general(
        x_ref[...], w_ref[...], (((1,), (0,)), ((), ())),
        preferred_element_type=jnp.float32,
        precision=lax.Precision.HIGHEST,
    )


def _dinv(degT):
    deg = jnp.sum(degT, axis=1, keepdims=True) + 1.0
    return jnp.where(deg > 0, lax.rsqrt(jnp.maximum(deg, 1e-12)), 0.0)


def _g_body(h_ref, degT_ref, o_ref):
    o_ref[...] = h_ref[...] * _dinv(degT_ref[...])


def _out_body(p_ref, g_ref, degT_ref, b_ref, o_ref):
    acc = p_ref[0] + p_ref[1] + g_ref[...]
    o_ref[...] = jnp.maximum(acc * _dinv(degT_ref[...]) + b_ref[...], 0.0)


def kernel(x, edge_index, edge_weight, W, b):
    n, d_in = x.shape
    d_out = W.shape[1]
    e = edge_weight.shape[0]

    n_pad = _round_up(n, NS * LANES * 8)
    e_pad = _round_up(e, NW * CB * 8)

    src = edge_index[0].astype(jnp.int32)
    dst = edge_index[1].astype(jnp.int32)
    srcp = jnp.pad(src, (0, e_pad - e))
    dstp = jnp.pad(dst, (0, e_pad - e))
    ewp = jnp.pad(edge_weight.astype(jnp.float32), (0, e_pad - e))
    x_pad = jnp.pad(x.astype(jnp.float32), ((0, n_pad - n), (0, 0)))
    b2 = b.astype(jnp.float32).reshape(1, d_out)

    n_tc = e_pad // CB
    ew_bits = lax.bitcast_convert_type(ewp, jnp.int32)
    epack = jnp.concatenate([
        srcp.reshape(n_tc, 1, CB),
        dstp.reshape(n_tc, 1, CB),
        ew_bits.reshape(n_tc, 1, CB),
        jnp.zeros((n_tc, 5, CB), jnp.int32),
    ], axis=1)

    deg_parts = _make_deg_kernel(n_pad, e_pad)(dstp, ewp)
    degT = deg_parts.reshape(NW, n_pad).T

    bm = 1024
    grid = (n_pad // bm,)

    h = pl.pallas_call(
        _mm_body,
        grid=grid,
        in_specs=[
            pl.BlockSpec((bm, d_in), lambda i: (i, 0)),
            pl.BlockSpec((d_in, d_out), lambda i: (0, 0)),
        ],
        out_specs=pl.BlockSpec((bm, d_out), lambda i: (i, 0)),
        out_shape=jax.ShapeDtypeStruct((n_pad, d_out), jnp.float32),
    )(x_pad, W.astype(jnp.float32))

    g = pl.pallas_call(
        _g_body,
        grid=grid,
        in_specs=[
            pl.BlockSpec((bm, d_out), lambda i: (i, 0)),
            pl.BlockSpec((bm, NW), lambda i: (i, 0)),
        ],
        out_specs=pl.BlockSpec((bm, d_out), lambda i: (i, 0)),
        out_shape=jax.ShapeDtypeStruct((n_pad, d_out), jnp.float32),
    )(h, degT)

    tq = e_pad // (NS * CB)
    q0 = (tq // 2) // EBUF * EBUF
    zeros_nd = jnp.zeros((n_pad, d_out), jnp.float32)
    parts = _make_agg_kernel(n_pad, e_pad, d_out, q0, tq - q0)(
        g, epack, zeros_nd)
    parts = parts.reshape(NC, n_pad, d_out)

    out = pl.pallas_call(
        _out_body,
        grid=grid,
        in_specs=[
            pl.BlockSpec((NC, bm, d_out), lambda i: (0, i, 0)),
            pl.BlockSpec((bm, d_out), lambda i: (i, 0)),
            pl.BlockSpec((bm, NW), lambda i: (i, 0)),
            pl.BlockSpec((1, d_out), lambda i: (0, 0)),
        ],
        out_specs=pl.BlockSpec((bm, d_out), lambda i: (i, 0)),
        out_shape=jax.ShapeDtypeStruct((n_pad, d_out), jnp.float32),
    )(parts, g, degT, b2)

    return out[:n]


# speedup vs baseline: 1.0005x; 1.0005x over previous
import dataclasses
import functools

import jax
import jax.numpy as jnp
from jax import lax
from jax.experimental import pallas as pl
from jax.experimental.pallas import tpu as pltpu
from jax.experimental.pallas import tpu_sc as plsc

NC = 2
NS = 16
LANES = 16
NW = NC * NS

CB = 64


def _sc_compiler_params():
    cp = pltpu.CompilerParams()
    if "needs_layout_passes" in pltpu.CompilerParams.__dataclass_fields__:
        cp = dataclasses.replace(cp, needs_layout_passes=False)
    return cp


def _round_up(a: int, m: int) -> int:
    return ((a + m - 1) // m) * m


def _make_deg_kernel(n_pad: int, e_pad: int):
    e_per_w = e_pad // NW

    def body(dst_hbm, ew_hbm, out_hbm, dst_v, ew_v, acc_v):
        c = lax.axis_index("c")
        s = lax.axis_index("s")
        wid = c * NS + s
        base = wid * e_per_w

        zero16 = jnp.zeros((LANES,), jnp.float32)

        @pl.loop(0, n_pad, step=LANES)
        def _(i):
            acc_v[pl.ds(i, LANES)] = zero16

        pltpu.sync_copy(dst_hbm.at[pl.ds(base, e_per_w)], dst_v)
        pltpu.sync_copy(ew_hbm.at[pl.ds(base, e_per_w)], ew_v)

        lane = lax.iota(jnp.int32, LANES)

        @pl.loop(0, e_per_w, step=LANES)
        def _(i):
            idx = dst_v[pl.ds(i, LANES)]
            w = ew_v[pl.ds(i, LANES)]
            for l in range(LANES):
                plsc.addupdate_scatter(acc_v, [idx], w, mask=lane == l)

        pltpu.sync_copy(acc_v, out_hbm.at[pl.ds(wid * n_pad, n_pad)])

    return pl.kernel(
        body,
        out_type=jax.ShapeDtypeStruct((NW * n_pad,), jnp.float32),
        mesh=plsc.VectorSubcoreMesh(core_axis_name="c", subcore_axis_name="s"),
        scratch_types=[
            pltpu.VMEM((e_per_w,), jnp.int32),
            pltpu.VMEM((e_per_w,), jnp.float32),
            pltpu.VMEM((n_pad,), jnp.float32),
        ],
        compiler_params=_sc_compiler_params(),
    )


NBUF = 4
EBUF = 8


def _make_agg_kernel(n_pad: int, e_pad: int, d: int, q0: int, q1: int):
    rows_per_t = n_pad // NS
    n_wb = rows_per_t // CB
    assert q0 % EBUF == 0 and q1 % EBUF == 0 and min(q0, q1) >= EBUF
    assert (q0 + q1) * NS * CB == e_pad

    def body(g_hbm, ep_hbm, z_hbm, out_hbm, ebuf, rows_v, acc_sh, *sems):
        gsem = sems[:NBUF]
        ssem = sems[NBUF:2 * NBUF]
        esem = sems[2 * NBUF:]
        c = lax.axis_index("c")
        s = lax.axis_index("s")
        row0 = s * rows_per_t
        n_chunks = jnp.where(c == 0, q0, q1)
        chunk0 = jnp.where(c == 0, s * q0, NS * q0 + s * q1)

        with jax.named_scope("agg_zero"):
            pltpu.sync_copy(z_hbm.at[pl.ds(row0, rows_per_t)],
                            acc_sh.at[pl.ds(row0, rows_per_t)])

            plsc.subcore_barrier()

        def eload(k, eb):
            pltpu.async_copy(ep_hbm.at[chunk0 + k], ebuf.at[eb], esem[eb])

        def ewait(k, eb):
            pltpu.make_async_copy(ep_hbm.at[chunk0 + k], ebuf.at[eb],
                                  esem[eb]).wait()

        def gload(eb, rb):
            pltpu.async_copy(g_hbm.at[ebuf.at[eb, 0]], rows_v.at[rb],
                             gsem[rb])

        def gwait(eb, rb):
            pltpu.make_async_copy(g_hbm.at[ebuf.at[eb, 0]], rows_v.at[rb],
                                  gsem[rb]).wait()

        def swait(eb, rb):
            pltpu.make_async_copy(rows_v.at[rb], acc_sh.at[ebuf.at[eb, 1]],
                                  ssem[rb]).wait()

        with jax.named_scope("agg_prime"):
            for k in range(EBUF - 2):
                eload(k, k)
            for k in range(2):
                ewait(k, k)
                gload(k, k)

        sc_main = jax.named_scope("agg_main")
        sc_main.__enter__()

        @pl.loop(0, n_chunks, step=EBUF)
        def _(j0):
            for b in range(EBUF):
                j = j0 + b
                rb = b % NBUF
                buf = rows_v.at[rb]
                gwait(b, rb)

                @pl.loop(0, CB, step=LANES)
                def _(i):
                    wi = ebuf[b, 2, pl.ds(i, LANES)]
                    w16 = plsc.bitcast(wi, jnp.float32)
                    for l in range(LANES):
                        w = w16[l]
                        for jj in range(d // LANES):
                            sl = pl.ds(jj * LANES, LANES)
                            buf[i + l, sl] = buf[i + l, sl] * w

                pltpu.async_copy(buf, acc_sh.at[ebuf.at[b, 1]], ssem[rb],
                                 add=True)

                rb2 = (b + 2) % NBUF
                eb2 = (b + 2) % EBUF
                eb6 = (b + 6) % EBUF

                @pl.when(j >= 2)
                def _():
                    swait(eb2, rb2)

                @pl.when(j + 6 < n_chunks)
                def _():
                    eload(j + 6, eb6)

                @pl.when(j + 2 < n_chunks)
                def _():
                    ewait(j + 2, eb2)
                    gload(eb2, rb2)

        sc_main.__exit__(None, None, None)

        with jax.named_scope("agg_wb"):
            swait(EBUF - 2, NBUF - 2)
            swait(EBUF - 1, NBUF - 1)

            plsc.subcore_barrier()

            out_base = c * n_pad + row0
            pltpu.sync_copy(acc_sh.at[pl.ds(row0, rows_per_t)],
                            out_hbm.at[pl.ds(out_base, rows_per_t)])

    return pl.kernel(
        body,
        out_type=jax.ShapeDtypeStruct((NC * n_pad, d), jnp.float32),
        mesh=plsc.VectorSubcoreMesh(core_axis_name="c", subcore_axis_name="s"),
        scratch_types=[
            pltpu.VMEM((EBUF, 8, CB), jnp.int32),
            pltpu.VMEM((NBUF, CB, d), jnp.float32),
            pltpu.VMEM_SHARED((n_pad, d), jnp.float32),
        ] + [pltpu.SemaphoreType.DMA] * (2 * NBUF + EBUF),
        compiler_params=_sc_compiler_params(),
    )


def _mm_body(x_ref, w_ref, o_ref):
    o_ref[...] = lax.dot_general(
        x_ref[...], w_ref[...], (((1,), (0,)), ((), ())),
        preferred_element_type=jnp.float32,
        precision=lax.Precision.HIGHEST,
    )


def _dinv(degT):
    deg = jnp.sum(degT, axis=1, keepdims=True) + 1.0
    return jnp.where(deg > 0, lax.rsqrt(jnp.maximum(deg, 1e-12)), 0.0)


def _g_body(h_ref, degT_ref, o_ref):
    o_ref[...] = h_ref[...] * _dinv(degT_ref[...])


def _out_body(p_ref, g_ref, degT_ref, b_ref, o_ref):
    acc = p_ref[0] + p_ref[1] + g_ref[...]
    o_ref[...] = jnp.maximum(acc * _dinv(degT_ref[...]) + b_ref[...], 0.0)


def kernel(x, edge_index, edge_weight, W, b):
    n, d_in = x.shape
    d_out = W.shape[1]
    e = edge_weight.shape[0]

    n_pad = _round_up(n, NS * LANES * 8)
    e_pad = _round_up(e, NW * CB * 8)

    src = edge_index[0].astype(jnp.int32)
    dst = edge_index[1].astype(jnp.int32)
    srcp = jnp.pad(src, (0, e_pad - e))
    dstp = jnp.pad(dst, (0, e_pad - e))
    ewp = jnp.pad(edge_weight.astype(jnp.float32), (0, e_pad - e))
    x_pad = jnp.pad(x.astype(jnp.float32), ((0, n_pad - n), (0, 0)))
    b2 = b.astype(jnp.float32).reshape(1, d_out)

    n_tc = e_pad // CB
    ew_bits = lax.bitcast_convert_type(ewp, jnp.int32)
    epack = jnp.concatenate([
        srcp.reshape(n_tc, 1, CB),
        dstp.reshape(n_tc, 1, CB),
        ew_bits.reshape(n_tc, 1, CB),
        jnp.zeros((n_tc, 5, CB), jnp.int32),
    ], axis=1)

    deg_parts = _make_deg_kernel(n_pad, e_pad)(dstp, ewp)
    degT = deg_parts.reshape(NW, n_pad).T

    bm = 1024
    grid = (n_pad // bm,)

    h = pl.pallas_call(
        _mm_body,
        grid=grid,
        in_specs=[
            pl.BlockSpec((bm, d_in), lambda i: (i, 0)),
            pl.BlockSpec((d_in, d_out), lambda i: (0, 0)),
        ],
        out_specs=pl.BlockSpec((bm, d_out), lambda i: (i, 0)),
        out_shape=jax.ShapeDtypeStruct((n_pad, d_out), jnp.float32),
    )(x_pad, W.astype(jnp.float32))

    g = pl.pallas_call(
        _g_body,
        grid=grid,
        in_specs=[
            pl.BlockSpec((bm, d_out), lambda i: (i, 0)),
            pl.BlockSpec((bm, NW), lambda i: (i, 0)),
        ],
        out_specs=pl.BlockSpec((bm, d_out), lambda i: (i, 0)),
        out_shape=jax.ShapeDtypeStruct((n_pad, d_out), jnp.float32),
    )(h, degT)

    tq = e_pad // (NS * CB)
    q0 = (tq // 2) // EBUF * EBUF
    zeros_nd = jnp.zeros((n_pad, d_out), jnp.float32)
    parts = _make_agg_kernel(n_pad, e_pad, d_out, q0, tq - q0)(
        g, epack, zeros_nd)
    parts = parts.reshape(NC, n_pad, d_out)

    out = pl.pallas_call(
        _out_body,
        grid=grid,
        in_specs=[
            pl.BlockSpec((NC, bm, d_out), lambda i: (0, i, 0)),
            pl.BlockSpec((bm, d_out), lambda i: (i, 0)),
            pl.BlockSpec((bm, NW), lambda i: (i, 0)),
            pl.BlockSpec((1, d_out), lambda i: (0, 0)),
        ],
        out_specs=pl.BlockSpec((bm, d_out), lambda i: (i, 0)),
        out_shape=jax.ShapeDtypeStruct((n_pad, d_out), jnp.float32),
    )(parts, g, degT, b2)

    return out[:n]


# speedup vs baseline: 2.3878x; 2.3867x over previous
import dataclasses
import functools

import jax
import jax.numpy as jnp
from jax import lax
from jax.experimental import pallas as pl
from jax.experimental.pallas import tpu as pltpu
from jax.experimental.pallas import tpu_sc as plsc

NC = 2
NS = 16
LANES = 16
NW = NC * NS

CB = 64


def _sc_compiler_params():
    cp = pltpu.CompilerParams()
    if "needs_layout_passes" in pltpu.CompilerParams.__dataclass_fields__:
        cp = dataclasses.replace(cp, needs_layout_passes=False)
    return cp


def _round_up(a: int, m: int) -> int:
    return ((a + m - 1) // m) * m


def _make_deg_kernel(n_pad: int, e_pad: int):
    e_per_w = e_pad // NW

    def body(dst_hbm, ew_hbm, out_hbm, dst_v, ew_v, acc_v):
        c = lax.axis_index("c")
        s = lax.axis_index("s")
        wid = c * NS + s
        base = wid * e_per_w

        zero16 = jnp.zeros((LANES,), jnp.float32)

        @pl.loop(0, n_pad, step=LANES)
        def _(i):
            acc_v[pl.ds(i, LANES)] = zero16

        pltpu.sync_copy(dst_hbm.at[pl.ds(base, e_per_w)], dst_v)
        pltpu.sync_copy(ew_hbm.at[pl.ds(base, e_per_w)], ew_v)

        lane = lax.iota(jnp.int32, LANES)

        @pl.loop(0, e_per_w, step=LANES)
        def _(i):
            idx = dst_v[pl.ds(i, LANES)]
            w = ew_v[pl.ds(i, LANES)]
            for l in range(LANES):
                plsc.addupdate_scatter(acc_v, [idx], w, mask=lane == l)

        pltpu.sync_copy(acc_v, out_hbm.at[pl.ds(wid * n_pad, n_pad)])

    return pl.kernel(
        body,
        out_type=jax.ShapeDtypeStruct((NW * n_pad,), jnp.float32),
        mesh=plsc.VectorSubcoreMesh(core_axis_name="c", subcore_axis_name="s"),
        scratch_types=[
            pltpu.VMEM((e_per_w,), jnp.int32),
            pltpu.VMEM((e_per_w,), jnp.float32),
            pltpu.VMEM((n_pad,), jnp.float32),
        ],
        compiler_params=_sc_compiler_params(),
    )


NBUF = 4
EBUF = 8


def _make_agg_kernel(n_pad: int, e_pad: int, d: int, q0: int, q1: int):
    rows_per_t = n_pad // NS
    n_wb = rows_per_t // CB
    assert q0 % EBUF == 0 and q1 % EBUF == 0 and min(q0, q1) >= EBUF
    assert (q0 + q1) * NS * CB == e_pad

    def body(g_hbm, ep_hbm, z_hbm, out_hbm, ebuf, rows_v, acc_sh, *sems):
        gsem = sems[:NBUF]
        ssem = sems[NBUF:2 * NBUF]
        esem = sems[2 * NBUF:]
        c = lax.axis_index("c")
        s = lax.axis_index("s")
        row0 = s * rows_per_t
        n_chunks = jnp.where(c == 0, q0, q1)
        chunk0 = jnp.where(c == 0, s * q0, NS * q0 + s * q1)

        with jax.named_scope("agg_zero"):
            pltpu.sync_copy(z_hbm.at[pl.ds(row0, rows_per_t)],
                            acc_sh.at[pl.ds(row0, rows_per_t)])

            plsc.subcore_barrier()

        def eload(k, eb):
            pltpu.async_copy(ep_hbm.at[chunk0 + k], ebuf.at[eb], esem[eb])

        def ewait(k, eb):
            pltpu.make_async_copy(ep_hbm.at[chunk0 + k], ebuf.at[eb],
                                  esem[eb]).wait()

        def gload(eb, rb):
            pltpu.async_copy(g_hbm.at[ebuf.at[eb, 0]], rows_v.at[rb],
                             gsem[rb])

        def gwait(eb, rb):
            pltpu.make_async_copy(g_hbm.at[ebuf.at[eb, 0]], rows_v.at[rb],
                                  gsem[rb]).wait()

        def swait(eb, rb):
            pltpu.make_async_copy(rows_v.at[rb], acc_sh.at[ebuf.at[eb, 1]],
                                  ssem[rb]).wait()

        with jax.named_scope("agg_prime"):
            for k in range(EBUF - 2):
                eload(k, k)
            for k in range(2):
                ewait(k, k)
                gload(k, k)

        sc_main = jax.named_scope("agg_main")
        sc_main.__enter__()

        @pl.loop(0, n_chunks, step=EBUF)
        def _(j0):
            for b in range(EBUF):
                j = j0 + b
                rb = b % NBUF
                buf = rows_v.at[rb]
                gwait(b, rb)

                @pl.loop(0, CB, step=LANES)
                def _(i):
                    wi = ebuf[b, 2, pl.ds(i, LANES)]
                    w16 = plsc.bitcast(wi, jnp.float32)
                    for l in range(LANES):
                        w = w16[l]
                        for jj in range(d // LANES):
                            sl = pl.ds(jj * LANES, LANES)
                            buf[i + l, sl] = buf[i + l, sl] * w

                pltpu.async_copy(buf, acc_sh.at[ebuf.at[b, 1]], ssem[rb],
                                 add=True)

                rb2 = (b + 2) % NBUF
                eb2 = (b + 2) % EBUF
                eb6 = (b + 6) % EBUF

                @pl.when(j >= 2)
                def _():
                    swait(eb2, rb2)

                @pl.when(j + 6 < n_chunks)
                def _():
                    eload(j + 6, eb6)

                @pl.when(j + 2 < n_chunks)
                def _():
                    ewait(j + 2, eb2)
                    gload(eb2, rb2)

        sc_main.__exit__(None, None, None)

        with jax.named_scope("agg_wb"):
            swait(EBUF - 2, NBUF - 2)
            swait(EBUF - 1, NBUF - 1)

            plsc.subcore_barrier()

            out_base = c * n_pad + row0
            pltpu.sync_copy(acc_sh.at[pl.ds(row0, rows_per_t)],
                            out_hbm.at[pl.ds(out_base, rows_per_t)])

    return pl.kernel(
        body,
        out_type=jax.ShapeDtypeStruct((NC * n_pad, d), jnp.float32),
        mesh=plsc.VectorSubcoreMesh(core_axis_name="c", subcore_axis_name="s"),
        scratch_types=[
            pltpu.VMEM((EBUF, 8, CB), jnp.int32),
            pltpu.VMEM((NBUF, CB, d), jnp.float32),
            pltpu.VMEM_SHARED((n_pad, d), jnp.float32),
        ] + [pltpu.SemaphoreType.DMA] * (2 * NBUF + EBUF),
        compiler_params=_sc_compiler_params(),
    )


def _mm_body(x_ref, w_ref, o_ref):
    o_ref[...] = lax.dot_general(
        x_ref[...], w_ref[...], (((1,), (0,)), ((), ())),
        preferred_element_type=jnp.float32,
        precision=lax.Precision.HIGHEST,
    )


def _dinv(degT):
    deg = jnp.sum(degT, axis=1, keepdims=True) + 1.0
    return jnp.where(deg > 0, lax.rsqrt(jnp.maximum(deg, 1e-12)), 0.0)


def _g_body(h_ref, degT_ref, o_ref):
    o_ref[...] = h_ref[...] * _dinv(degT_ref[...])


def _out_body(p_ref, g_ref, degT_ref, b_ref, o_ref):
    acc = p_ref[0] + p_ref[1] + g_ref[...]
    o_ref[...] = jnp.maximum(acc * _dinv(degT_ref[...]) + b_ref[...], 0.0)


def kernel(x, edge_index, edge_weight, W, b):
    n, d_in = x.shape
    d_out = W.shape[1]
    e = edge_weight.shape[0]

    n_pad = _round_up(n, NS * LANES * 8)
    e_pad = _round_up(e, NW * CB * 8)

    src = edge_index[0].astype(jnp.int32)
    dst = edge_index[1].astype(jnp.int32)
    pad_idx = (jnp.arange(e_pad - e, dtype=jnp.int32) % (n_pad - n)) + n
    srcp = jnp.concatenate([src, pad_idx])
    dstp = jnp.concatenate([dst, pad_idx])
    ewp = jnp.pad(edge_weight.astype(jnp.float32), (0, e_pad - e))
    x_pad = jnp.pad(x.astype(jnp.float32), ((0, n_pad - n), (0, 0)))
    b2 = b.astype(jnp.float32).reshape(1, d_out)

    n_tc = e_pad // CB
    ew_bits = lax.bitcast_convert_type(ewp, jnp.int32)
    epack = jnp.concatenate([
        srcp.reshape(n_tc, 1, CB),
        dstp.reshape(n_tc, 1, CB),
        ew_bits.reshape(n_tc, 1, CB),
        jnp.zeros((n_tc, 5, CB), jnp.int32),
    ], axis=1)

    deg_parts = _make_deg_kernel(n_pad, e_pad)(dstp, ewp)
    degT = deg_parts.reshape(NW, n_pad).T

    bm = 1024
    grid = (n_pad // bm,)

    h = pl.pallas_call(
        _mm_body,
        grid=grid,
        in_specs=[
            pl.BlockSpec((bm, d_in), lambda i: (i, 0)),
            pl.BlockSpec((d_in, d_out), lambda i: (0, 0)),
        ],
        out_specs=pl.BlockSpec((bm, d_out), lambda i: (i, 0)),
        out_shape=jax.ShapeDtypeStruct((n_pad, d_out), jnp.float32),
    )(x_pad, W.astype(jnp.float32))

    g = pl.pallas_call(
        _g_body,
        grid=grid,
        in_specs=[
            pl.BlockSpec((bm, d_out), lambda i: (i, 0)),
            pl.BlockSpec((bm, NW), lambda i: (i, 0)),
        ],
        out_specs=pl.BlockSpec((bm, d_out), lambda i: (i, 0)),
        out_shape=jax.ShapeDtypeStruct((n_pad, d_out), jnp.float32),
    )(h, degT)

    tq = e_pad // (NS * CB)
    q0 = (tq // 2) // EBUF * EBUF
    zeros_nd = jnp.zeros((n_pad, d_out), jnp.float32)
    parts = _make_agg_kernel(n_pad, e_pad, d_out, q0, tq - q0)(
        g, epack, zeros_nd)
    parts = parts.reshape(NC, n_pad, d_out)

    out = pl.pallas_call(
        _out_body,
        grid=grid,
        in_specs=[
            pl.BlockSpec((NC, bm, d_out), lambda i: (0, i, 0)),
            pl.BlockSpec((bm, d_out), lambda i: (i, 0)),
            pl.BlockSpec((bm, NW), lambda i: (i, 0)),
            pl.BlockSpec((1, d_out), lambda i: (0, 0)),
        ],
        out_specs=pl.BlockSpec((bm, d_out), lambda i: (i, 0)),
        out_shape=jax.ShapeDtypeStruct((n_pad, d_out), jnp.float32),
    )(parts, g, degT, b2)

    return out[:n]
